# baseline (speedup 1.0000x reference)
import jax
import jax.numpy as jnp
from jax import lax
from jax.experimental import pallas as pl
from jax.experimental.pallas import tpu as pltpu
from jax.experimental.pallas import tpu_sc as plsc

_N = 10000
_E = 160000
_D = 256
_R = 32
_HOPS = 3

_NC = 2
_NS = 16
_CE = 64
_EPAD = 163840
_NH = 8
_UR = _EPAD // _NS // _CE // _NH
_NSS = 10240
_STR = _NSS // _NS
_PF = _N * _R + 16


def _tc_prep_body(emb, rel, w, v1_o, v2_o, p1_o, p2_o, esp_o):
    v = lax.dot_general(w[...], rel[...], (((1,), (1,)), ((), ())),
                        preferred_element_type=jnp.float32)
    v1 = v[:_D]
    v2 = v[_D:]
    v1_o[...] = v1
    v2_o[...] = v2
    e = emb[...]
    p1_o[...] = jnp.dot(e, v1, preferred_element_type=jnp.float32)
    p2_o[...] = jnp.dot(e, v2, preferred_element_type=jnp.float32)
    esp_o[...] = jnp.stack([e[:, :128], e[:, 128:]])


def _tc_hop_body(agg2, ssum, emb_p, res_p, v1, v2,
                 emb_o, res_o, p1_o, p2_o, esp_o):
    a = agg2[...]
    ss2 = ssum[...]
    agg = jnp.concatenate([a[0], a[1]], axis=1) / (ss2[0] + ss2[1] + 1e-16)
    cat = agg + emb_p[...]
    nrm = jnp.maximum(
        jnp.sqrt(jnp.sum(cat * cat, axis=1, keepdims=True)), 1e-12)
    e = cat / nrm
    emb_o[...] = e
    res_o[...] = 0.5 * res_p[...] + e
    p1_o[...] = jnp.dot(e, v1[...], preferred_element_type=jnp.float32)
    p2_o[...] = jnp.dot(e, v2[...], preferred_element_type=jnp.float32)
    esp_o[...] = jnp.stack([e[:, :128], e[:, 128:]])


_BN = 2000
_GRID = _N // _BN

_tc_prep = pl.pallas_call(
    _tc_prep_body,
    grid=(_GRID,),
    in_specs=[
        pl.BlockSpec((_BN, _D), lambda i: (i, 0)),
        pl.BlockSpec((_R, _D), lambda i: (0, 0)),
        pl.BlockSpec((2 * _D, _D), lambda i: (0, 0)),
    ],
    out_specs=[
        pl.BlockSpec((_D, _R), lambda i: (0, 0)),
        pl.BlockSpec((_D, _R), lambda i: (0, 0)),
        pl.BlockSpec((_BN, _R), lambda i: (i, 0)),
        pl.BlockSpec((_BN, _R), lambda i: (i, 0)),
        pl.BlockSpec((2, _BN, 128), lambda i: (0, i, 0)),
    ],
    out_shape=[
        jax.ShapeDtypeStruct((_D, _R), jnp.float32),
        jax.ShapeDtypeStruct((_D, _R), jnp.float32),
        jax.ShapeDtypeStruct((_N, _R), jnp.float32),
        jax.ShapeDtypeStruct((_N, _R), jnp.float32),
        jax.ShapeDtypeStruct((2, _N, 128), jnp.float32),
    ],
)

_tc_hop = pl.pallas_call(
    _tc_hop_body,
    grid=(_GRID,),
    in_specs=[
        pl.BlockSpec((2, _BN, 128), lambda i: (0, i, 0)),
        pl.BlockSpec((2, _BN, 1), lambda i: (0, i, 0)),
        pl.BlockSpec((_BN, _D), lambda i: (i, 0)),
        pl.BlockSpec((_BN, _D), lambda i: (i, 0)),
        pl.BlockSpec((_D, _R), lambda i: (0, 0)),
        pl.BlockSpec((_D, _R), lambda i: (0, 0)),
    ],
    out_specs=[
        pl.BlockSpec((_BN, _D), lambda i: (i, 0)),
        pl.BlockSpec((_BN, _D), lambda i: (i, 0)),
        pl.BlockSpec((_BN, _R), lambda i: (i, 0)),
        pl.BlockSpec((_BN, _R), lambda i: (i, 0)),
        pl.BlockSpec((2, _BN, 128), lambda i: (0, i, 0)),
    ],
    out_shape=[
        jax.ShapeDtypeStruct((_N, _D), jnp.float32),
        jax.ShapeDtypeStruct((_N, _D), jnp.float32),
        jax.ShapeDtypeStruct((_N, _R), jnp.float32),
        jax.ShapeDtypeStruct((_N, _R), jnp.float32),
        jax.ShapeDtypeStruct((2, _N, 128), jnp.float32),
    ],
)


def _sc_hop_body(embcat, pf, pr3, h3, t4, agg_o, ss_o,
                 h2, t2, pr2,
                 ga, gb, r0, r1, r2, r3, abuf_a, abuf_b,
                 ssp, aggs,
                 sP0, sP1, sE0, sE1, sR0, sR1, sR2, sR3,
                 sS0, sS1, sS2, sS3):
    c = lax.axis_index("c")
    s = lax.axis_index("s")
    zv = jnp.zeros((16,), jnp.float32)
    rbufs = [r0, r1, r2, r3]
    sR = [sR0, sR1, sR2, sR3]
    sS = [sS0, sS1, sS2, sS3]
    gp = [ga, gb]
    abuf = [abuf_a, abuf_b]
    sP = [sP0, sP1]
    sE = [sE0, sE1]

    def _zr(i, carry):
        for q in range(8):
            r0[i, pl.ds(q * 16, 16)] = zv
        return carry

    lax.fori_loop(0, _CE, _zr, 0)
    for q in range(4):
        ga[pl.ds(q * 16, 16)] = zv
    for k in range(_STR // _CE):
        pltpu.sync_copy(ga.at[pl.ds(0, _CE)],
                        ssp.at[pl.ds(s * _STR + k * _CE, _CE)])
        pltpu.sync_copy(r0, aggs.at[pl.ds(s * _STR + k * _CE, _CE)])

    plsc.subcore_barrier()

    def _drain_rows(buf, sem):
        pltpu.make_async_copy(embcat.at[pl.ds(0, _CE)], buf, sem).wait()

    def _drain_vec(buf, sem):
        pltpu.make_async_copy(pf.at[pl.ds(0, _CE)], buf, sem).wait()

    def _drain_vec2(buf, sem):
        pltpu.make_async_copy(pf.at[pl.ds(0, 2 * _CE)], buf, sem).wait()

    def _eexp(gx, ab):
        for q in range(4):
            sl = pl.ds(q * 16, 16)
            v = gx[pl.ds(q * 16, 16)] + gx[pl.ds(_CE + q * 16, 16)]
            v = jnp.where(v >= 0.0, v, 0.2 * v)
            ab[sl] = jnp.exp(v)

    def _mul(rows, ab):
        def body(g, cc):
            a16 = ab[pl.ds(g * 16, 16)]
            for k in range(16):
                av = jnp.full((16,), a16[k], jnp.float32)
                for q in range(8):
                    sl = pl.ds(q * 16, 16)
                    rows[g * 16 + k, sl] = rows[g * 16 + k, sl] * av
            return cc

        lax.fori_loop(0, _CE // 16, body, 0)

    def _unit(u, w, first, last):
        b = rbufs[w]
        a = w % 2
        if not (first and w < 2):
            _drain_rows(rbufs[(w + 2) % 4], sS[(w + 2) % 4])
        if not (last and w >= 2):
            pltpu.async_copy(embcat.at[t2.at[u + 2]], rbufs[(w + 2) % 4],
                             sR[(w + 2) % 4])
        _drain_vec2(gp[a], sP[a])
        if not (first and w < 2):
            @pl.when(c == a)
            def _():
                _drain_vec(abuf[a], sE[a])
        _eexp(gp[a], abuf[a])

        @pl.when(c == a)
        def _():
            pltpu.async_copy(abuf[a], ssp.at[h2.at[u]], sE[a], add=True)

        if not (last and w >= 2):
            pltpu.async_copy(pf.at[pr2.at[u + 2]], gp[a], sP[a])
        _drain_rows(b, sR[w])
        _mul(b, abuf[a])
        pltpu.async_copy(b, aggs.at[h2.at[u]], sS[w], add=True)

    def _quad(q, first, last):
        for w in range(4):
            _unit(4 * q + w, w, first, last)

    def _round(rnd, cc):
        pltpu.sync_copy(pr3.at[s, rnd], pr2)
        pltpu.sync_copy(h3.at[s, rnd], h2)
        pltpu.sync_copy(t4.at[c, s, rnd], t2)
        pltpu.async_copy(embcat.at[t2.at[0]], r0, sR0)
        pltpu.async_copy(embcat.at[t2.at[1]], r1, sR1)
        pltpu.async_copy(pf.at[pr2.at[0]], ga, sP0)
        pltpu.async_copy(pf.at[pr2.at[1]], gb, sP1)
        _quad(0, True, False)

        def _mid(k, cc2):
            _quad(k, False, False)
            return cc2

        lax.fori_loop(1, _UR // 4 - 1, _mid, 0)
        _quad(_UR // 4 - 1, False, True)
        _drain_rows(r2, sS2)
        _drain_rows(r3, sS3)

        @pl.when(c == 0)
        def _():
            _drain_vec(abuf_a, sE0)

        @pl.when(c == 1)
        def _():
            _drain_vec(abuf_b, sE1)

        return cc

    lax.fori_loop(0, _NH, _round, 0)
    plsc.subcore_barrier()

    pltpu.sync_copy(aggs.at[pl.ds(s * _STR, _STR)],
                    agg_o.at[c, pl.ds(s * _STR, _STR)])
    pltpu.sync_copy(ssp.at[pl.ds(s * _STR, _STR)],
                    ss_o.at[c, pl.ds(s * _STR, _STR)])


def _make_sc_hop():
    mesh = plsc.VectorSubcoreMesh(core_axis_name="c", subcore_axis_name="s",
                                  num_cores=_NC, num_subcores=_NS)
    return pl.kernel(
        _sc_hop_body,
        out_type=[
            jax.ShapeDtypeStruct((_NC, _NSS, 128), jnp.float32),
            jax.ShapeDtypeStruct((_NC, _NSS), jnp.float32),
        ],
        mesh=mesh,
        scratch_types=[
            pltpu.VMEM((_UR, _CE), jnp.int32),
            pltpu.VMEM((_UR, _CE), jnp.int32),
            pltpu.VMEM((_UR, 2 * _CE), jnp.int32),
            pltpu.VMEM((2 * _CE,), jnp.float32),
            pltpu.VMEM((2 * _CE,), jnp.float32),
            pltpu.VMEM((_CE, 128), jnp.float32),
            pltpu.VMEM((_CE, 128), jnp.float32),
            pltpu.VMEM((_CE, 128), jnp.float32),
            pltpu.VMEM((_CE, 128), jnp.float32),
            pltpu.VMEM((_CE,), jnp.float32),
            pltpu.VMEM((_CE,), jnp.float32),
            pltpu.VMEM_SHARED((_NSS,), jnp.float32),
            pltpu.VMEM_SHARED((_NSS, 128), jnp.float32),
        ] + [pltpu.SemaphoreType.DMA] * 12,
    )


_sc_hop = _make_sc_hop()


def kernel(edge_index, edge_type, entity_emb, relation_emb, W):
    head = edge_index[0]
    tail = edge_index[1]
    i32 = jnp.int32
    hr = head * _R + edge_type
    tr = tail * _R + edge_type
    pad = _EPAD - _E
    hp = jnp.concatenate([head, jnp.full((pad,), _N, i32)])
    tp = jnp.concatenate([tail, jnp.zeros((pad,), i32)])
    hrp = jnp.concatenate([hr, jnp.full((pad,), _N * _R, i32)])
    trp = jnp.concatenate([tr, jnp.full((pad,), _N * _R, i32)])
    h3 = hp.reshape(_NS, _NH, _UR, _CE)
    hr4 = hrp.reshape(_NS, _NH, _UR, _CE)
    tr4 = trp.reshape(_NS, _NH, _UR, _CE) + _PF
    pr3 = jnp.concatenate([hr4, tr4], axis=-1)
    t4 = jnp.stack([tp, tp + _N]).reshape(_NC, _NS, _NH, _UR, _CE)

    v1, v2, p1, p2, esp = _tc_prep(entity_emb, relation_emb, W)
    emb = entity_emb
    res = entity_emb
    zt = jnp.zeros((16,), jnp.float32)
    for _hop in range(_HOPS):
        pf = jnp.concatenate([p1.reshape(-1), zt, p2.reshape(-1), zt])
        embcat = esp.reshape(_NC * _N, 128)
        agg2, ss = _sc_hop(embcat, pf, pr3, h3, t4)
        ssum = ss[:, :_N].reshape(_NC, _N, 1)
        emb, res, p1, p2, esp = _tc_hop(agg2[:, :_N], ssum, emb, res, v1, v2)
    return res


# speedup vs baseline: 1.0240x; 1.0240x over previous
import jax
import jax.numpy as jnp
from jax import lax
from jax.experimental import pallas as pl
from jax.experimental.pallas import tpu as pltpu
from jax.experimental.pallas import tpu_sc as plsc

_N = 10000
_E = 160000
_D = 256
_R = 32
_HOPS = 3

_NC = 2
_NS = 16
_CE = 64
_EPAD = 163840
_NH = 8
_UR = _EPAD // _NS // _CE // _NH
_NSS = 10240
_STR = _NSS // _NS
_PF = _N * _R + 16


def _tc_prep_body(emb, rel, w, v1_o, v2_o, p1_o, p2_o, esp_o):
    v = lax.dot_general(w[...], rel[...], (((1,), (1,)), ((), ())),
                        preferred_element_type=jnp.float32)
    v1 = v[:_D]
    v2 = v[_D:]
    v1_o[...] = v1
    v2_o[...] = v2
    e = emb[...]
    p1_o[...] = jnp.dot(e, v1, preferred_element_type=jnp.float32)
    p2_o[...] = jnp.dot(e, v2, preferred_element_type=jnp.float32)
    esp_o[...] = jnp.stack([e[:, :128], e[:, 128:]])


def _tc_hop_body(agg2, ssum, emb_p, res_p, v1, v2,
                 emb_o, res_o, p1_o, p2_o, esp_o):
    a = agg2[...]
    ss2 = ssum[...]
    agg = jnp.concatenate([a[0], a[1]], axis=1) / (ss2[0] + ss2[1] + 1e-16)
    cat = agg + emb_p[...]
    nrm = jnp.maximum(
        jnp.sqrt(jnp.sum(cat * cat, axis=1, keepdims=True)), 1e-12)
    e = cat / nrm
    emb_o[...] = e
    res_o[...] = 0.5 * res_p[...] + e
    p1_o[...] = jnp.dot(e, v1[...], preferred_element_type=jnp.float32)
    p2_o[...] = jnp.dot(e, v2[...], preferred_element_type=jnp.float32)
    esp_o[...] = jnp.stack([e[:, :128], e[:, 128:]])


_BN = 2000
_GRID = _N // _BN

_tc_prep = pl.pallas_call(
    _tc_prep_body,
    grid=(_GRID,),
    in_specs=[
        pl.BlockSpec((_BN, _D), lambda i: (i, 0)),
        pl.BlockSpec((_R, _D), lambda i: (0, 0)),
        pl.BlockSpec((2 * _D, _D), lambda i: (0, 0)),
    ],
    out_specs=[
        pl.BlockSpec((_D, _R), lambda i: (0, 0)),
        pl.BlockSpec((_D, _R), lambda i: (0, 0)),
        pl.BlockSpec((_BN, _R), lambda i: (i, 0)),
        pl.BlockSpec((_BN, _R), lambda i: (i, 0)),
        pl.BlockSpec((2, _BN, 128), lambda i: (0, i, 0)),
    ],
    out_shape=[
        jax.ShapeDtypeStruct((_D, _R), jnp.float32),
        jax.ShapeDtypeStruct((_D, _R), jnp.float32),
        jax.ShapeDtypeStruct((_N, _R), jnp.float32),
        jax.ShapeDtypeStruct((_N, _R), jnp.float32),
        jax.ShapeDtypeStruct((2, _N, 128), jnp.float32),
    ],
)

_tc_hop = pl.pallas_call(
    _tc_hop_body,
    grid=(_GRID,),
    in_specs=[
        pl.BlockSpec((2, _BN, 128), lambda i: (0, i, 0)),
        pl.BlockSpec((2, _BN, 1), lambda i: (0, i, 0)),
        pl.BlockSpec((_BN, _D), lambda i: (i, 0)),
        pl.BlockSpec((_BN, _D), lambda i: (i, 0)),
        pl.BlockSpec((_D, _R), lambda i: (0, 0)),
        pl.BlockSpec((_D, _R), lambda i: (0, 0)),
    ],
    out_specs=[
        pl.BlockSpec((_BN, _D), lambda i: (i, 0)),
        pl.BlockSpec((_BN, _D), lambda i: (i, 0)),
        pl.BlockSpec((_BN, _R), lambda i: (i, 0)),
        pl.BlockSpec((_BN, _R), lambda i: (i, 0)),
        pl.BlockSpec((2, _BN, 128), lambda i: (0, i, 0)),
    ],
    out_shape=[
        jax.ShapeDtypeStruct((_N, _D), jnp.float32),
        jax.ShapeDtypeStruct((_N, _D), jnp.float32),
        jax.ShapeDtypeStruct((_N, _R), jnp.float32),
        jax.ShapeDtypeStruct((_N, _R), jnp.float32),
        jax.ShapeDtypeStruct((2, _N, 128), jnp.float32),
    ],
)


def _sc_hop_body(embcat, p1f, p2f, hr3, tr3, h3, t4, agg_o, ss_o,
                 h2, t2, hr2, tr2,
                 g1a, g2a, g1b, g2b, r0, r1, r2, r3, abuf_a, abuf_b,
                 ssp, aggs,
                 sP0, sP1, sE0, sE1, sR0, sR1, sR2, sR3,
                 sS0, sS1, sS2, sS3):
    c = lax.axis_index("c")
    s = lax.axis_index("s")
    zv = jnp.zeros((16,), jnp.float32)
    rbufs = [r0, r1, r2, r3]
    sR = [sR0, sR1, sR2, sR3]
    sS = [sS0, sS1, sS2, sS3]
    g1 = [g1a, g1b]
    g2 = [g2a, g2b]
    abuf = [abuf_a, abuf_b]
    sP = [sP0, sP1]
    sE = [sE0, sE1]

    def _zr(i, carry):
        for q in range(8):
            r0[i, pl.ds(q * 16, 16)] = zv
        return carry

    lax.fori_loop(0, _CE, _zr, 0)
    for q in range(4):
        g1a[pl.ds(q * 16, 16)] = zv
    for k in range(_STR // _CE):
        pltpu.sync_copy(g1a, ssp.at[pl.ds(s * _STR + k * _CE, _CE)])
        pltpu.sync_copy(r0, aggs.at[pl.ds(s * _STR + k * _CE, _CE)])

    plsc.subcore_barrier()

    def _drain_rows(buf, sem):
        pltpu.make_async_copy(embcat.at[pl.ds(0, _CE)], buf, sem).wait()

    def _drain_vec(buf, sem):
        pltpu.make_async_copy(p1f.at[pl.ds(0, _CE)], buf, sem).wait()

    def _eexp(gx, gy, ab):
        for q in range(4):
            sl = pl.ds(q * 16, 16)
            v = gx[sl] + gy[sl]
            v = jnp.where(v >= 0.0, v, 0.2 * v)
            ab[sl] = jnp.exp(v)

    def _mul(rows, ab):
        def body(g, cc):
            a16 = ab[pl.ds(g * 16, 16)]
            for k in range(16):
                av = jnp.full((16,), a16[k], jnp.float32)
                for q in range(8):
                    sl = pl.ds(q * 16, 16)
                    rows[g * 16 + k, sl] = rows[g * 16 + k, sl] * av
            return cc

        lax.fori_loop(0, _CE // 16, body, 0)

    def _unit(u, w, first, last):
        b = rbufs[w]
        a = w % 2
        if not (first and w < 2):
            _drain_rows(rbufs[(w + 2) % 4], sS[(w + 2) % 4])
        if not (last and w >= 2):
            pltpu.async_copy(embcat.at[t2.at[u + 2]], rbufs[(w + 2) % 4],
                             sR[(w + 2) % 4])
        _drain_vec(g1[a], sP[a])
        _drain_vec(g2[a], sP[a])
        if not (first and w < 2):
            @pl.when(c == a)
            def _():
                _drain_vec(abuf[a], sE[a])
        _eexp(g1[a], g2[a], abuf[a])

        @pl.when(c == a)
        def _():
            pltpu.async_copy(abuf[a], ssp.at[h2.at[u]], sE[a], add=True)

        if not (last and w >= 2):
            pltpu.async_copy(p1f.at[hr2.at[u + 2]], g1[a], sP[a])
            pltpu.async_copy(p2f.at[tr2.at[u + 2]], g2[a], sP[a])
        _drain_rows(b, sR[w])
        _mul(b, abuf[a])
        pltpu.async_copy(b, aggs.at[h2.at[u]], sS[w], add=True)

    def _quad(q, first, last):
        for w in range(4):
            _unit(4 * q + w, w, first, last)

    def _round(rnd, cc):
        pltpu.sync_copy(hr3.at[s, rnd], hr2)
        pltpu.sync_copy(tr3.at[s, rnd], tr2)
        pltpu.sync_copy(h3.at[s, rnd], h2)
        pltpu.sync_copy(t4.at[c, s, rnd], t2)
        pltpu.async_copy(embcat.at[t2.at[0]], r0, sR0)
        pltpu.async_copy(embcat.at[t2.at[1]], r1, sR1)
        pltpu.async_copy(p1f.at[hr2.at[0]], g1a, sP0)
        pltpu.async_copy(p2f.at[tr2.at[0]], g2a, sP0)
        pltpu.async_copy(p1f.at[hr2.at[1]], g1b, sP1)
        pltpu.async_copy(p2f.at[tr2.at[1]], g2b, sP1)
        _quad(0, True, False)

        def _mid(k, cc2):
            _quad(k, False, False)
            return cc2

        lax.fori_loop(1, _UR // 4 - 1, _mid, 0)
        _quad(_UR // 4 - 1, False, True)
        _drain_rows(r2, sS2)
        _drain_rows(r3, sS3)

        @pl.when(c == 0)
        def _():
            _drain_vec(abuf_a, sE0)

        @pl.when(c == 1)
        def _():
            _drain_vec(abuf_b, sE1)

        return cc

    lax.fori_loop(0, _NH, _round, 0)
    plsc.subcore_barrier()

    pltpu.sync_copy(aggs.at[pl.ds(s * _STR, _STR)],
                    agg_o.at[c, pl.ds(s * _STR, _STR)])
    pltpu.sync_copy(ssp.at[pl.ds(s * _STR, _STR)],
                    ss_o.at[c, pl.ds(s * _STR, _STR)])


def _make_sc_hop():
    mesh = plsc.VectorSubcoreMesh(core_axis_name="c", subcore_axis_name="s",
                                  num_cores=_NC, num_subcores=_NS)
    return pl.kernel(
        _sc_hop_body,
        out_type=[
            jax.ShapeDtypeStruct((_NC, _NSS, 128), jnp.float32),
            jax.ShapeDtypeStruct((_NC, _NSS), jnp.float32),
        ],
        mesh=mesh,
        scratch_types=[
            pltpu.VMEM((_UR, _CE), jnp.int32),
            pltpu.VMEM((_UR, _CE), jnp.int32),
            pltpu.VMEM((_UR, _CE), jnp.int32),
            pltpu.VMEM((_UR, _CE), jnp.int32),
            pltpu.VMEM((_CE,), jnp.float32),
            pltpu.VMEM((_CE,), jnp.float32),
            pltpu.VMEM((_CE,), jnp.float32),
            pltpu.VMEM((_CE,), jnp.float32),
            pltpu.VMEM((_CE, 128), jnp.float32),
            pltpu.VMEM((_CE, 128), jnp.float32),
            pltpu.VMEM((_CE, 128), jnp.float32),
            pltpu.VMEM((_CE, 128), jnp.float32),
            pltpu.VMEM((_CE,), jnp.float32),
            pltpu.VMEM((_CE,), jnp.float32),
            pltpu.VMEM_SHARED((_NSS,), jnp.float32),
            pltpu.VMEM_SHARED((_NSS, 128), jnp.float32),
        ] + [pltpu.SemaphoreType.DMA] * 12,
    )


_sc_hop = _make_sc_hop()


def kernel(edge_index, edge_type, entity_emb, relation_emb, W):
    head = edge_index[0]
    tail = edge_index[1]
    i32 = jnp.int32
    hr = head * _R + edge_type
    tr = tail * _R + edge_type
    pad = _EPAD - _E
    hp = jnp.concatenate([head, jnp.full((pad,), _N, i32)])
    tp = jnp.concatenate([tail, jnp.zeros((pad,), i32)])
    hrp = jnp.concatenate([hr, jnp.full((pad,), _N * _R, i32)])
    trp = jnp.concatenate([tr, jnp.full((pad,), _N * _R, i32)])
    h3 = hp.reshape(_NS, _NH, _UR, _CE)
    hr3 = hrp.reshape(_NS, _NH, _UR, _CE)
    tr3 = trp.reshape(_NS, _NH, _UR, _CE)
    t4 = jnp.stack([tp, tp + _N]).reshape(_NC, _NS, _NH, _UR, _CE)

    v1, v2, p1, p2, esp = _tc_prep(entity_emb, relation_emb, W)
    emb = entity_emb
    res = entity_emb
    zt = jnp.zeros((16,), jnp.float32)
    for _hop in range(_HOPS):
        p1f = jnp.concatenate([p1.reshape(-1), zt])
        p2f = jnp.concatenate([p2.reshape(-1), zt])
        embcat = esp.reshape(_NC * _N, 128)
        agg2, ss = _sc_hop(embcat, p1f, p2f, hr3, tr3, h3, t4)
        ssum = ss[:, :_N].reshape(_NC, _N, 1)
        emb, res, p1, p2, esp = _tc_hop(agg2[:, :_N], ssum, emb, res, v1, v2)
    return res


# speedup vs baseline: 1.0980x; 1.0723x over previous
import jax
import jax.numpy as jnp
from jax import lax
from jax.experimental import pallas as pl
from jax.experimental.pallas import tpu as pltpu
from jax.experimental.pallas import tpu_sc as plsc

_N = 10000
_E = 160000
_D = 256
_R = 32
_HOPS = 3

_NC = 2
_NS = 16
_CE = 64
_EPAD = 163840
_NH = 8
_UR = _EPAD // _NS // _CE // _NH
_NSS = 10240
_STR = _NSS // _NS
_PF = _N * _R + 16


def _tc_prep_body(emb, rel, w, v1_o, v2_o, p1_o, p2_o, esp_o):
    v = lax.dot_general(w[...], rel[...], (((1,), (1,)), ((), ())),
                        preferred_element_type=jnp.float32)
    v1 = v[:_D]
    v2 = v[_D:]
    v1_o[...] = v1
    v2_o[...] = v2
    e = emb[...]
    p1_o[...] = jnp.dot(e, v1, preferred_element_type=jnp.float32)
    p2_o[...] = jnp.dot(e, v2, preferred_element_type=jnp.float32)
    esp_o[...] = jnp.stack([e[:, :128], e[:, 128:]])


def _tc_hop_body(agg2, ssum, emb_p, res_p, v1, v2,
                 emb_o, res_o, p1_o, p2_o, esp_o):
    a = agg2[...]
    agg = jnp.concatenate([a[0], a[1]], axis=1) / (ssum[...] + 1e-16)
    cat = agg + emb_p[...]
    nrm = jnp.maximum(
        jnp.sqrt(jnp.sum(cat * cat, axis=1, keepdims=True)), 1e-12)
    e = cat / nrm
    emb_o[...] = e
    res_o[...] = 0.5 * res_p[...] + e
    p1_o[...] = jnp.dot(e, v1[...], preferred_element_type=jnp.float32)
    p2_o[...] = jnp.dot(e, v2[...], preferred_element_type=jnp.float32)
    esp_o[...] = jnp.stack([e[:, :128], e[:, 128:]])


_BN = 2000
_GRID = _N // _BN

_tc_prep = pl.pallas_call(
    _tc_prep_body,
    grid=(_GRID,),
    in_specs=[
        pl.BlockSpec((_BN, _D), lambda i: (i, 0)),
        pl.BlockSpec((_R, _D), lambda i: (0, 0)),
        pl.BlockSpec((2 * _D, _D), lambda i: (0, 0)),
    ],
    out_specs=[
        pl.BlockSpec((_D, _R), lambda i: (0, 0)),
        pl.BlockSpec((_D, _R), lambda i: (0, 0)),
        pl.BlockSpec((_BN, _R), lambda i: (i, 0)),
        pl.BlockSpec((_BN, _R), lambda i: (i, 0)),
        pl.BlockSpec((2, _BN, 128), lambda i: (0, i, 0)),
    ],
    out_shape=[
        jax.ShapeDtypeStruct((_D, _R), jnp.float32),
        jax.ShapeDtypeStruct((_D, _R), jnp.float32),
        jax.ShapeDtypeStruct((_N, _R), jnp.float32),
        jax.ShapeDtypeStruct((_N, _R), jnp.float32),
        jax.ShapeDtypeStruct((2, _N, 128), jnp.float32),
    ],
)

_tc_hop = pl.pallas_call(
    _tc_hop_body,
    grid=(_GRID,),
    in_specs=[
        pl.BlockSpec((2, _BN, 128), lambda i: (0, i, 0)),
        pl.BlockSpec((_BN, 1), lambda i: (i, 0)),
        pl.BlockSpec((_BN, _D), lambda i: (i, 0)),
        pl.BlockSpec((_BN, _D), lambda i: (i, 0)),
        pl.BlockSpec((_D, _R), lambda i: (0, 0)),
        pl.BlockSpec((_D, _R), lambda i: (0, 0)),
    ],
    out_specs=[
        pl.BlockSpec((_BN, _D), lambda i: (i, 0)),
        pl.BlockSpec((_BN, _D), lambda i: (i, 0)),
        pl.BlockSpec((_BN, _R), lambda i: (i, 0)),
        pl.BlockSpec((_BN, _R), lambda i: (i, 0)),
        pl.BlockSpec((2, _BN, 128), lambda i: (0, i, 0)),
    ],
    out_shape=[
        jax.ShapeDtypeStruct((_N, _D), jnp.float32),
        jax.ShapeDtypeStruct((_N, _D), jnp.float32),
        jax.ShapeDtypeStruct((_N, _R), jnp.float32),
        jax.ShapeDtypeStruct((_N, _R), jnp.float32),
        jax.ShapeDtypeStruct((2, _N, 128), jnp.float32),
    ],
)


def _sc_hop_body(embcat, p1f, p2f, hr3, tr3, h3, t4, agg_o, ss_o,
                 h2, t2, hr2, tr2,
                 g1a, g2a, g1b, g2b, r0, r1, r2, r3, abuf_a, abuf_b,
                 ssp, aggs,
                 sP0, sP1, sE0, sE1, sR0, sR1, sR2, sR3,
                 sS0, sS1, sS2, sS3):
    c = lax.axis_index("c")
    s = lax.axis_index("s")
    zv = jnp.zeros((16,), jnp.float32)
    rbufs = [r0, r1, r2, r3]
    sR = [sR0, sR1, sR2, sR3]
    sS = [sS0, sS1, sS2, sS3]
    g1 = [g1a, g1b]
    g2 = [g2a, g2b]
    abuf = [abuf_a, abuf_b]
    sP = [sP0, sP1]
    sE = [sE0, sE1]

    def _zr(i, carry):
        for q in range(8):
            r0[i, pl.ds(q * 16, 16)] = zv
        return carry

    lax.fori_loop(0, _CE, _zr, 0)
    for q in range(4):
        g1a[pl.ds(q * 16, 16)] = zv
    for k in range(_STR // _CE):
        pltpu.sync_copy(g1a, ssp.at[pl.ds(s * _STR + k * _CE, _CE)])
        pltpu.sync_copy(r0, aggs.at[pl.ds(s * _STR + k * _CE, _CE)])

    plsc.subcore_barrier()

    def _drain_rows(buf, sem):
        pltpu.make_async_copy(embcat.at[pl.ds(0, _CE)], buf, sem).wait()

    def _drain_vec(buf, sem):
        pltpu.make_async_copy(p1f.at[pl.ds(0, _CE)], buf, sem).wait()

    def _eexp(gx, gy, ab):
        for q in range(4):
            sl = pl.ds(q * 16, 16)
            v = gx[sl] + gy[sl]
            v = jnp.where(v >= 0.0, v, 0.2 * v)
            ab[sl] = jnp.exp(v)

    def _mul(rows, ab):
        def body(g, cc):
            a16 = ab[pl.ds(g * 16, 16)]
            for k in range(16):
                av = jnp.full((16,), a16[k], jnp.float32)
                for q in range(8):
                    sl = pl.ds(q * 16, 16)
                    rows[g * 16 + k, sl] = rows[g * 16 + k, sl] * av
            return cc

        lax.fori_loop(0, _CE // 16, body, 0)

    def _unit(u, w, first, last):
        b = rbufs[w]
        a = w % 2
        if not (first and w < 2):
            _drain_rows(rbufs[(w + 2) % 4], sS[(w + 2) % 4])
        if not (last and w >= 2):
            pltpu.async_copy(embcat.at[t2.at[u + 2]], rbufs[(w + 2) % 4],
                             sR[(w + 2) % 4])
        _drain_vec(g1[a], sP[a])
        _drain_vec(g2[a], sP[a])
        if not (first and w < 2):
            _drain_vec(abuf[a], sE[a])
        _eexp(g1[a], g2[a], abuf[a])
        pltpu.async_copy(abuf[a], ssp.at[h2.at[u]], sE[a], add=True)
        if not (last and w >= 2):
            pltpu.async_copy(p1f.at[hr2.at[u + 2]], g1[a], sP[a])
            pltpu.async_copy(p2f.at[tr2.at[u + 2]], g2[a], sP[a])
        _drain_rows(b, sR[w])
        _mul(b, abuf[a])
        pltpu.async_copy(b, aggs.at[h2.at[u]], sS[w], add=True)

    def _quad(q, first, last):
        for w in range(4):
            _unit(4 * q + w, w, first, last)

    def _round(rnd, cc):
        pltpu.sync_copy(hr3.at[s, rnd], hr2)
        pltpu.sync_copy(tr3.at[s, rnd], tr2)
        pltpu.sync_copy(h3.at[s, rnd], h2)
        pltpu.sync_copy(t4.at[c, s, rnd], t2)
        pltpu.async_copy(embcat.at[t2.at[0]], r0, sR0)
        pltpu.async_copy(embcat.at[t2.at[1]], r1, sR1)
        pltpu.async_copy(p1f.at[hr2.at[0]], g1a, sP0)
        pltpu.async_copy(p2f.at[tr2.at[0]], g2a, sP0)
        pltpu.async_copy(p1f.at[hr2.at[1]], g1b, sP1)
        pltpu.async_copy(p2f.at[tr2.at[1]], g2b, sP1)
        _quad(0, True, False)

        def _mid(k, cc2):
            _quad(k, False, False)
            return cc2

        lax.fori_loop(1, _UR // 4 - 1, _mid, 0)
        _quad(_UR // 4 - 1, False, True)
        _drain_rows(r2, sS2)
        _drain_rows(r3, sS3)
        _drain_vec(abuf_a, sE0)
        _drain_vec(abuf_b, sE1)
        return cc

    lax.fori_loop(0, _NH, _round, 0)
    plsc.subcore_barrier()

    pltpu.sync_copy(aggs.at[pl.ds(s * _STR, _STR)],
                    agg_o.at[c, pl.ds(s * _STR, _STR)])
    pltpu.sync_copy(ssp.at[pl.ds(s * _STR, _STR)],
                    ss_o.at[c, pl.ds(s * _STR, _STR)])


def _make_sc_hop():
    mesh = plsc.VectorSubcoreMesh(core_axis_name="c", subcore_axis_name="s",
                                  num_cores=_NC, num_subcores=_NS)
    return pl.kernel(
        _sc_hop_body,
        out_type=[
            jax.ShapeDtypeStruct((_NC, _NSS, 128), jnp.float32),
            jax.ShapeDtypeStruct((_NC, _NSS), jnp.float32),
        ],
        mesh=mesh,
        scratch_types=[
            pltpu.VMEM((_UR, _CE), jnp.int32),
            pltpu.VMEM((_UR, _CE), jnp.int32),
            pltpu.VMEM((_UR, _CE), jnp.int32),
            pltpu.VMEM((_UR, _CE), jnp.int32),
            pltpu.VMEM((_CE,), jnp.float32),
            pltpu.VMEM((_CE,), jnp.float32),
            pltpu.VMEM((_CE,), jnp.float32),
            pltpu.VMEM((_CE,), jnp.float32),
            pltpu.VMEM((_CE, 128), jnp.float32),
            pltpu.VMEM((_CE, 128), jnp.float32),
            pltpu.VMEM((_CE, 128), jnp.float32),
            pltpu.VMEM((_CE, 128), jnp.float32),
            pltpu.VMEM((_CE,), jnp.float32),
            pltpu.VMEM((_CE,), jnp.float32),
            pltpu.VMEM_SHARED((_NSS,), jnp.float32),
            pltpu.VMEM_SHARED((_NSS, 128), jnp.float32),
        ] + [pltpu.SemaphoreType.DMA] * 12,
    )


_sc_hop = _make_sc_hop()


def kernel(edge_index, edge_type, entity_emb, relation_emb, W):
    head = edge_index[0]
    tail = edge_index[1]
    i32 = jnp.int32
    hr = head * _R + edge_type
    tr = tail * _R + edge_type
    pad = _EPAD - _E
    hp = jnp.concatenate([head, jnp.full((pad,), _N, i32)])
    tp = jnp.concatenate([tail, jnp.zeros((pad,), i32)])
    hrp = jnp.concatenate([hr, jnp.full((pad,), _N * _R, i32)])
    trp = jnp.concatenate([tr, jnp.full((pad,), _N * _R, i32)])
    h3 = hp.reshape(_NS, _NH, _UR, _CE)
    hr3 = hrp.reshape(_NS, _NH, _UR, _CE)
    tr3 = trp.reshape(_NS, _NH, _UR, _CE)
    t4 = jnp.stack([tp, tp + _N]).reshape(_NC, _NS, _NH, _UR, _CE)

    v1, v2, p1, p2, esp = _tc_prep(entity_emb, relation_emb, W)
    emb = entity_emb
    res = entity_emb
    zt = jnp.zeros((16,), jnp.float32)
    for _hop in range(_HOPS):
        p1f = jnp.concatenate([p1.reshape(-1), zt])
        p2f = jnp.concatenate([p2.reshape(-1), zt])
        embcat = esp.reshape(_NC * _N, 128)
        agg2, ss = _sc_hop(embcat, p1f, p2f, hr3, tr3, h3, t4)
        ssum = ss[0, :_N].reshape(_N, 1)
        emb, res, p1, p2, esp = _tc_hop(agg2[:, :_N], ssum, emb, res, v1, v2)
    return res
